# Initial kernel scaffold; baseline (speedup 1.0000x reference)
import jax
import jax.numpy as jnp
from jax.experimental import pallas as pl


def kernel(x, edge_index, W_embed, b_embed, W_attn, b_attn, W_conv, b_conv):
    raise NotImplementedError("write your pallas kernel here")


# speedup vs baseline: 8.0013x; 8.0013x over previous
import functools

import jax
import jax.numpy as jnp
from jax import lax
from jax.experimental import pallas as pl
from jax.experimental.pallas import tpu as pltpu
from jax.experimental.pallas import tpu_sc as plsc

_NC, _NS, _L = 2, 16, 16
_NW = _NC * _NS
_CH = 8

_BCAST_DNUMS = lax.GatherDimensionNumbers(
    offset_dims=(), collapsed_slice_dims=(0,), start_index_map=(0,))


def _lane_bcast(v, kk):
  idx = jnp.full((_L, 1), kk, jnp.int32)
  return lax.gather(v, idx, _BCAST_DNUMS, (1,),
                    mode=lax.GatherScatterMode.PROMISE_IN_BOUNDS)


def _attn_scalars_tc(xt, W_embed, wa2, be2, ba11):
  Np, C = xt.shape

  def body(xt_ref, we_ref, wa_ref, be_ref, ba_ref, o_ref):
    U = jnp.dot(wa_ref[...], we_ref[...],
                preferred_element_type=jnp.float32)
    c = jnp.sum(wa_ref[...] * be_ref[...], axis=1,
                keepdims=True)
    badd = jnp.concatenate(
        [ba_ref[...], jnp.zeros((1, 1), jnp.float32)], axis=1)
    o_ref[...] = (jnp.dot(xt_ref[...], U.T,
                          preferred_element_type=jnp.float32)
                  + c.T + badd)

  return pl.pallas_call(
      body,
      out_shape=jax.ShapeDtypeStruct((Np, 2), jnp.float32),
  )(xt, W_embed, wa2, be2, ba11)


def _sc_aggregate(xt, gidx, alpha, beta, nchunk):
  Np, C = xt.shape
  npw = nchunk * _CH
  mesh = plsc.VectorSubcoreMesh(
      core_axis_name="c", subcore_axis_name="s",
      num_cores=_NC, num_subcores=_NS)

  @functools.partial(
      pl.kernel,
      out_type=jax.ShapeDtypeStruct((Np, C), jnp.float32),
      mesh=mesh,
      compiler_params=pltpu.CompilerParams(needs_layout_passes=False),
      scratch_types=[
          pltpu.VMEM((Np,), jnp.float32),
          pltpu.VMEM((nchunk, _CH * _L), jnp.int32),
          pltpu.VMEM((npw,), jnp.float32),
          pltpu.VMEM((_CH * _L, C), jnp.float32),
          pltpu.VMEM((_CH, C), jnp.float32),
          pltpu.SemaphoreType.DMA,
      ],
  )
  def k(xt_hbm, gidx_hbm, alpha_hbm, beta_hbm, out_hbm,
        beta_v, idx_v, alpha_v, rows_v, agg_v, sem):
    wid = lax.axis_index("s") * _NC + lax.axis_index("c")
    pltpu.sync_copy(beta_hbm, beta_v)
    pltpu.sync_copy(gidx_hbm.at[wid], idx_v)
    pltpu.sync_copy(alpha_hbm.at[wid], alpha_v)

    def chunk_body(j, carry):
      pltpu.async_copy(xt_hbm.at[idx_v.at[j]], rows_v, sem).wait()
      for i in range(_CH):
        idxv = idx_v[j, pl.ds(i * _L, _L)]
        betav = plsc.load_gather(beta_v, [idxv])
        n_loc = j * _CH + i
        alphav = plsc.load_gather(
            alpha_v, [jnp.zeros((_L,), jnp.int32) + n_loc])
        z = alphav + betav
        lg = jnp.maximum(z, 0.1 * z)
        m = jnp.max(lg)
        e = jnp.exp(lg - m)
        s = jnp.sum(e)
        w = e / s
        wbs = [_lane_bcast(w, kk) for kk in range(_L)]
        for co in range(C // _L):
          acc = wbs[0] * rows_v[i * _L, pl.ds(co * _L, _L)]
          for kk in range(1, _L):
            acc = acc + wbs[kk] * rows_v[i * _L + kk, pl.ds(co * _L, _L)]
          agg_v[i, pl.ds(co * _L, _L)] = acc
      pltpu.sync_copy(agg_v, out_hbm.at[pl.ds(wid * npw + j * _CH, _CH)])
      return carry

    lax.fori_loop(0, nchunk, chunk_body, 0)

  return k(xt, gidx, alpha, beta)


def _conv_tc(xt, agg, W1t, W2t, b2):
  Np, C = xt.shape
  blk = 2048
  grid = Np // blk

  def body(xt_ref, agg_ref, w1_ref, w2_ref, b_ref, o_ref):
    h = (jnp.dot(xt_ref[...], w1_ref[...],
                 preferred_element_type=jnp.float32)
         + jnp.dot(agg_ref[...], w2_ref[...],
                   preferred_element_type=jnp.float32)
         + b_ref[...])
    o_ref[...] = jnp.maximum(h, 0.0) + xt_ref[...]

  return pl.pallas_call(
      body,
      grid=(grid,),
      in_specs=[
          pl.BlockSpec((blk, C), lambda i: (i, 0)),
          pl.BlockSpec((blk, C), lambda i: (i, 0)),
          pl.BlockSpec((C, C), lambda i: (0, 0)),
          pl.BlockSpec((C, C), lambda i: (0, 0)),
          pl.BlockSpec((1, C), lambda i: (0, 0)),
      ],
      out_specs=pl.BlockSpec((blk, C), lambda i: (i, 0)),
      out_shape=jax.ShapeDtypeStruct((Np, C), jnp.float32),
  )(xt, agg, W1t, W2t, b2)


def kernel(x, edge_index, W_embed, b_embed, W_attn, b_attn, W_conv, b_conv):
  B, C, N, _ = x.shape
  K = edge_index.shape[-1]
  n_nodes = B * N
  npw = -(-n_nodes // (_NW * 64)) * 64
  nchunk = npw // _CH
  Np = _NW * npw

  xt = jnp.transpose(x[..., 0], (0, 2, 1)).reshape(n_nodes, C)
  xt_p = jnp.pad(xt, ((0, Np - n_nodes), (0, 0)))
  gidx = (edge_index[0].astype(jnp.int32)
          + (jnp.arange(B, dtype=jnp.int32) * N)[:, None, None])
  gidx_p = jnp.pad(gidx.reshape(n_nodes * K),
                   (0, (Np - n_nodes) * K)).reshape(_NW, nchunk, _CH * _L)

  wa2 = jnp.concatenate([W_attn[:, :C], W_attn[:, C:]], axis=0)
  ab = _attn_scalars_tc(xt_p, W_embed, wa2, b_embed[None, :],
                        b_attn.reshape(1, 1))
  alpha = ab[:, 0].reshape(_NW, npw)
  beta = ab[:, 1]

  agg = _sc_aggregate(xt_p, gidx_p, alpha, beta, nchunk)

  out = _conv_tc(xt_p, agg, W_conv[:, :C].T, W_conv[:, C:].T,
                 b_conv[None, :])
  h = out[:n_nodes].reshape(B, N, C)
  return jnp.transpose(h, (0, 2, 1))[..., None]


# speedup vs baseline: 9.0611x; 1.1325x over previous
import functools

import jax
import jax.numpy as jnp
from jax import lax
from jax.experimental import pallas as pl
from jax.experimental.pallas import tpu as pltpu
from jax.experimental.pallas import tpu_sc as plsc

_NC, _NS, _L = 2, 16, 16
_NW = _NC * _NS
_CH = 8

_BCAST_DNUMS = lax.GatherDimensionNumbers(
    offset_dims=(), collapsed_slice_dims=(0,), start_index_map=(0,))


def _lane_bcast(v, kk):
  idx = jnp.full((_L, 1), kk, jnp.int32)
  return lax.gather(v, idx, _BCAST_DNUMS, (1,),
                    mode=lax.GatherScatterMode.PROMISE_IN_BOUNDS)


def _attn_scalars_tc(xt, W_embed, wa2, be2, ba11):
  Np, C = xt.shape

  def body(xt_ref, we_ref, wa_ref, be_ref, ba_ref, o_ref):
    U = jnp.dot(wa_ref[...], we_ref[...],
                preferred_element_type=jnp.float32)
    c = jnp.sum(wa_ref[...] * be_ref[...], axis=1,
                keepdims=True)
    badd = jnp.concatenate(
        [ba_ref[...], jnp.zeros((1, 1), jnp.float32)], axis=1)
    o_ref[...] = (jnp.dot(xt_ref[...], U.T,
                          preferred_element_type=jnp.float32)
                  + c.T + badd)

  return pl.pallas_call(
      body,
      out_shape=jax.ShapeDtypeStruct((Np, 2), jnp.float32),
  )(xt, W_embed, wa2, be2, ba11)


def _sc_aggregate(xt, gidx, alpha, beta, nchunk):
  Np, C = xt.shape
  npw = nchunk * _CH
  mesh = plsc.VectorSubcoreMesh(
      core_axis_name="c", subcore_axis_name="s",
      num_cores=_NC, num_subcores=_NS)

  @functools.partial(
      pl.kernel,
      out_type=jax.ShapeDtypeStruct((Np, C), jnp.float32),
      mesh=mesh,
      compiler_params=pltpu.CompilerParams(needs_layout_passes=False),
      scratch_types=[
          pltpu.VMEM((Np,), jnp.float32),
          pltpu.VMEM((nchunk, _CH * _L), jnp.int32),
          pltpu.VMEM((npw,), jnp.float32),
          pltpu.VMEM((2, _CH * _L, C), jnp.float32),
          pltpu.VMEM((2, _CH, C), jnp.float32),
          pltpu.SemaphoreType.DMA,
          pltpu.SemaphoreType.DMA,
          pltpu.SemaphoreType.DMA,
          pltpu.SemaphoreType.DMA,
      ],
  )
  def k(xt_hbm, gidx_hbm, alpha_hbm, beta_hbm, out_hbm,
        beta_v, idx_v, alpha_v, rows_v, agg_v, g0, g1, o0, o1):
    wid = lax.axis_index("s") * _NC + lax.axis_index("c")
    gsem = (g0, g1)
    osem = (o0, o1)
    pltpu.sync_copy(beta_hbm, beta_v)
    pltpu.sync_copy(gidx_hbm.at[wid], idx_v)
    pltpu.sync_copy(alpha_hbm.at[wid], alpha_v)

    def gather_desc(j, b):
      return pltpu.make_async_copy(
          xt_hbm.at[idx_v.at[j]], rows_v.at[b], gsem[b])

    def out_desc(j, b):
      return pltpu.make_async_copy(
          agg_v.at[b], out_hbm.at[pl.ds(wid * npw + j * _CH, _CH)], osem[b])

    gather_desc(0, 0).start()
    gather_desc(1, 1).start()

    def pair_body(jj, carry):
      for b in range(2):
        j = jj * 2 + b
        gather_desc(j, b).wait()

        @pl.when(jj >= 1)
        def _wait_out():
          out_desc(j - 2, b).wait()

        for i in range(_CH):
          idxv = idx_v[j, pl.ds(i * _L, _L)]
          betav = plsc.load_gather(beta_v, [idxv])
          n_loc = j * _CH + i
          alphav = plsc.load_gather(
              alpha_v, [jnp.zeros((_L,), jnp.int32) + n_loc])
          z = alphav + betav
          lg = jnp.maximum(z, 0.1 * z)
          m = jnp.max(lg)
          e = jnp.exp(lg - m)
          s = jnp.sum(e)
          w = e / s
          wbs = [_lane_bcast(w, kk) for kk in range(_L)]
          for co in range(C // _L):
            acc = wbs[0] * rows_v[b, i * _L, pl.ds(co * _L, _L)]
            for kk in range(1, _L):
              acc = acc + wbs[kk] * rows_v[b, i * _L + kk,
                                           pl.ds(co * _L, _L)]
            agg_v[b, i, pl.ds(co * _L, _L)] = acc
        out_desc(j, b).start()

        @pl.when(j + 2 < nchunk)
        def _prefetch():
          gather_desc(j + 2, b).start()
      return carry

    lax.fori_loop(0, nchunk // 2, pair_body, 0)
    out_desc(nchunk - 2, 0).wait()
    out_desc(nchunk - 1, 1).wait()

  return k(xt, gidx, alpha, beta)


def _conv_tc(xt, agg, W1t, W2t, b2):
  Np, C = xt.shape
  blk = 2048
  grid = Np // blk

  def body(xt_ref, agg_ref, w1_ref, w2_ref, b_ref, o_ref):
    h = (jnp.dot(xt_ref[...], w1_ref[...],
                 preferred_element_type=jnp.float32)
         + jnp.dot(agg_ref[...], w2_ref[...],
                   preferred_element_type=jnp.float32)
         + b_ref[...])
    o_ref[...] = jnp.maximum(h, 0.0) + xt_ref[...]

  return pl.pallas_call(
      body,
      grid=(grid,),
      in_specs=[
          pl.BlockSpec((blk, C), lambda i: (i, 0)),
          pl.BlockSpec((blk, C), lambda i: (i, 0)),
          pl.BlockSpec((C, C), lambda i: (0, 0)),
          pl.BlockSpec((C, C), lambda i: (0, 0)),
          pl.BlockSpec((1, C), lambda i: (0, 0)),
      ],
      out_specs=pl.BlockSpec((blk, C), lambda i: (i, 0)),
      out_shape=jax.ShapeDtypeStruct((Np, C), jnp.float32),
  )(xt, agg, W1t, W2t, b2)


def kernel(x, edge_index, W_embed, b_embed, W_attn, b_attn, W_conv, b_conv):
  B, C, N, _ = x.shape
  K = edge_index.shape[-1]
  n_nodes = B * N
  npw = -(-n_nodes // (_NW * 64)) * 64
  nchunk = npw // _CH
  Np = _NW * npw

  xt = jnp.transpose(x[..., 0], (0, 2, 1)).reshape(n_nodes, C)
  xt_p = jnp.pad(xt, ((0, Np - n_nodes), (0, 0)))
  gidx = (edge_index[0].astype(jnp.int32)
          + (jnp.arange(B, dtype=jnp.int32) * N)[:, None, None])
  gidx_p = jnp.pad(gidx.reshape(n_nodes * K),
                   (0, (Np - n_nodes) * K)).reshape(_NW, nchunk, _CH * _L)

  wa2 = jnp.concatenate([W_attn[:, :C], W_attn[:, C:]], axis=0)
  ab = _attn_scalars_tc(xt_p, W_embed, wa2, b_embed[None, :],
                        b_attn.reshape(1, 1))
  alpha = ab[:, 0].reshape(_NW, npw)
  beta = ab[:, 1]

  agg = _sc_aggregate(xt_p, gidx_p, alpha, beta, nchunk)

  out = _conv_tc(xt_p, agg, W_conv[:, :C].T, W_conv[:, C:].T,
                 b_conv[None, :])
  h = out[:n_nodes].reshape(B, N, C)
  return jnp.transpose(h, (0, 2, 1))[..., None]
